# baseline (speedup 1.0000x reference)
import jax
import jax.numpy as jnp
from jax import lax
from jax.experimental import pallas as pl
from jax.experimental.pallas import tpu as pltpu
from jax.experimental.pallas import tpu_sc as plsc

_VOCAB = 100000
_EMB = 32
_MAXW = 7
_B = 1024
_L = 200
_NCORES = 2
_NSUB = 16
_NW = _NCORES * _NSUB
_ROWS_PER_W = _B // _NW
_CHUNKS = ((0, 128), (128, 72))
_L3 = _L - 2
_L5 = _L - 4
_L7 = _L - 6
_VB = 8192
_ROW = _MAXW * _EMB + _EMB


def _pack_body(kt_ref, wt_ref, out_ref):
    t = jnp.transpose(kt_ref[...], (1, 0))
    w = jnp.transpose(wt_ref[...], (1, 0))
    out_ref[...] = jnp.concatenate([t, w], axis=1)


def _sc_body(seq_hbm, t_hbm, o3_hbm, o5_hbm, o7_hbm,
             idx0, idx1, dst0, dst1, o3_v, o5_v, o7_v, gsem, ssem):
    cid = lax.axis_index("c")
    sid = lax.axis_index("s")
    wid = sid * _NCORES + cid
    bufs = ((idx0, dst0), (idx1, dst1))

    def gather_copies(p):
        idx_v, dst = bufs[p]
        cps = []
        for off, n in _CHUNKS:
            sl = pl.ds(off, n)
            cps.append(pltpu.make_async_copy(
                t_hbm.at[idx_v.at[sl]], dst.at[sl], gsem))
        return cps

    def issue_g(ri, p):
        idx_v, dst = bufs[p]
        b = wid * _ROWS_PER_W + ri
        pltpu.sync_copy(seq_hbm.at[b], idx_v)
        for cp in gather_copies(p):
            cp.start()

    def wait_g(p):
        for cp in gather_copies(p):
            cp.wait()

    def scatter_copies(ri):
        b = wid * _ROWS_PER_W + ri
        return [pltpu.make_async_copy(o3_v, o3_hbm.at[b], ssem),
                pltpu.make_async_copy(o5_v, o5_hbm.at[b], ssem),
                pltpu.make_async_copy(o7_v, o7_hbm.at[b], ssem)]

    def issue_s(ri):
        for cp in scatter_copies(ri):
            cp.start()

    def wait_s():
        for cp in scatter_copies(0):
            cp.wait()

    def compute(p):
        idx_v, dst = bufs[p]

        def wrow(c, h):
            return dst[c, pl.ds(_MAXW * _EMB + h * 16, 16)]

        def kv(c, d, h):
            return dst[c, pl.ds((3 + d) * _EMB + h * 16, 16)]

        def _p(c, d, h):
            return wrow(c + d, h) * kv(c, d, h)

        init = tuple(wrow(c, h) for c in range(6) for h in range(2))

        @plsc.parallel_loop(3, _L - 3, carry=init, unroll=2)
        def c_body(c, win):
            new = (wrow(c + 3, 0), wrow(c + 3, 1))
            rows = tuple(win[2 * i:2 * i + 2] for i in range(6)) + (new,)
            for h in range(2):
                hs = pl.ds(h * 16, 16)
                pr = [rows[3 + d][h] * kv(c, d, h) for d in range(-3, 4)]
                m3 = jnp.maximum(jnp.maximum(pr[2], pr[3]), pr[4])
                m5 = jnp.maximum(m3, jnp.maximum(pr[1], pr[5]))
                m7 = jnp.maximum(m5, jnp.maximum(pr[0], pr[6]))
                o3_v[c - 1, hs] = m3
                o5_v[c - 2, hs] = m5
                o7_v[c - 3, hs] = m7
            return win[2:] + new

        for c in (1, 2, _L - 3, _L - 2):
            for h in range(2):
                hs = pl.ds(h * 16, 16)
                p0 = _p(c, 0, h)
                m3 = jnp.maximum(jnp.maximum(_p(c, -1, h), p0), _p(c, 1, h))
                o3_v[c - 1, hs] = m3
                if 2 <= c <= _L - 3:
                    m5 = jnp.maximum(
                        m3, jnp.maximum(_p(c, -2, h), _p(c, 2, h)))
                    o5_v[c - 2, hs] = m5

    issue_g(0, 0)
    wait_g(0)
    issue_g(1, 1)
    compute(0)
    issue_s(0)

    def pair_body(i, carry):
        ri = 1 + 2 * i
        for step in range(2):
            p = (1 + step) % 2
            wait_g(p)
            issue_g(ri + step + 1, 1 - p)
            wait_s()
            compute(p)
            issue_s(ri + step)
        return carry

    lax.fori_loop(0, (_ROWS_PER_W - 2) // 2, pair_body, None)

    wait_g(1)
    wait_s()
    compute(1)
    issue_s(_ROWS_PER_W - 1)
    wait_s()


@jax.jit
def _impl(seq, W, K):
    seq2 = seq.astype(jnp.int32)
    kt = jnp.transpose(K, (1, 2, 0)).reshape(_MAXW * _EMB, _VOCAB)
    wt = jnp.transpose(W, (1, 0))
    grid = (_VOCAB + _VB - 1) // _VB
    table = pl.pallas_call(
        _pack_body,
        grid=(grid,),
        in_specs=[
            pl.BlockSpec((_MAXW * _EMB, _VB), lambda i: (0, i)),
            pl.BlockSpec((_EMB, _VB), lambda i: (0, i)),
        ],
        out_specs=pl.BlockSpec((_VB, _ROW), lambda i: (i, 0)),
        out_shape=jax.ShapeDtypeStruct((_VOCAB, _ROW), jnp.float32),
    )(kt, wt)

    mesh = plsc.VectorSubcoreMesh(core_axis_name="c", subcore_axis_name="s")
    run = pl.kernel(
        _sc_body,
        mesh=mesh,
        compiler_params=pltpu.CompilerParams(use_tc_tiling_on_sc=False),
        out_type=(
            jax.ShapeDtypeStruct((_B, _L3, _EMB), jnp.float32),
            jax.ShapeDtypeStruct((_B, _L5, _EMB), jnp.float32),
            jax.ShapeDtypeStruct((_B, _L7, _EMB), jnp.float32),
        ),
        scratch_types=[
            pltpu.VMEM((_L,), jnp.int32),
            pltpu.VMEM((_L,), jnp.int32),
            pltpu.VMEM((_L, _ROW), jnp.float32),
            pltpu.VMEM((_L, _ROW), jnp.float32),
            pltpu.VMEM((_L3, _EMB), jnp.float32),
            pltpu.VMEM((_L5, _EMB), jnp.float32),
            pltpu.VMEM((_L7, _EMB), jnp.float32),
            pltpu.SemaphoreType.DMA,
            pltpu.SemaphoreType.DMA,
        ],
    )
    o3, o5, o7 = run(seq2, table)
    zero = lax.optimization_barrier(jnp.zeros((), jnp.int32))

    def _tc(o):
        return lax.bitcast_convert_type(
            lax.bitcast_convert_type(o, jnp.int32) | zero, jnp.float32)

    return _tc(o3), _tc(o5), _tc(o7)


def kernel(seq, W, K):
    return _impl(seq, W, K)


# speedup vs baseline: 1.3969x; 1.3969x over previous
import jax
import jax.numpy as jnp
from jax import lax
from jax.experimental import pallas as pl
from jax.experimental.pallas import tpu as pltpu
from jax.experimental.pallas import tpu_sc as plsc

_VOCAB = 100000
_EMB = 32
_MAXW = 7
_B = 1024
_L = 200
_NCORES = 2
_NSUB = 16
_NW = _NCORES * _NSUB
_ROWS_PER_W = _B // _NW
_CHUNKS = ((0, 128), (128, 72))
_L3 = _L - 2
_L5 = _L - 4
_L7 = _L - 6
_VB = 8192
_ROW = _MAXW * _EMB + _EMB


def _pack_body(kt_ref, wt_ref, out_ref):
    t = jnp.transpose(kt_ref[...], (1, 0))
    w = jnp.transpose(wt_ref[...], (1, 0))
    out_ref[...] = jnp.concatenate([t, w], axis=1)


def _sc_body(seq_hbm, t_hbm, o3_hbm, o5_hbm, o7_hbm,
             idx0, idx1, dst0, dst1, o3_v, o5_v, o7_v, gsem, ssem):
    cid = lax.axis_index("c")
    sid = lax.axis_index("s")
    wid = sid * _NCORES + cid
    bufs = ((idx0, dst0), (idx1, dst1))

    def gather_copies(p):
        idx_v, dst = bufs[p]
        cps = []
        for off, n in _CHUNKS:
            sl = pl.ds(off, n)
            cps.append(pltpu.make_async_copy(
                t_hbm.at[idx_v.at[sl]], dst.at[sl], gsem))
        return cps

    def issue_g(ri, p):
        idx_v, dst = bufs[p]
        b = wid * _ROWS_PER_W + ri
        pltpu.sync_copy(seq_hbm.at[b], idx_v)
        for cp in gather_copies(p):
            cp.start()

    def wait_g(p):
        for cp in gather_copies(p):
            cp.wait()

    def scatter_copies(ri):
        b = wid * _ROWS_PER_W + ri
        return [pltpu.make_async_copy(o3_v, o3_hbm.at[b], ssem),
                pltpu.make_async_copy(o5_v, o5_hbm.at[b], ssem),
                pltpu.make_async_copy(o7_v, o7_hbm.at[b], ssem)]

    def issue_s(ri):
        for cp in scatter_copies(ri):
            cp.start()

    def wait_s():
        for cp in scatter_copies(0):
            cp.wait()

    def compute(p):
        idx_v, dst = bufs[p]

        def wrow(c, h):
            return dst[c, pl.ds(_MAXW * _EMB + h * 16, 16)]

        def kv(c, d, h):
            return dst[c, pl.ds((3 + d) * _EMB + h * 16, 16)]

        def _p(c, d, h):
            return wrow(c + d, h) * kv(c, d, h)

        init = tuple(wrow(c, h) for c in range(6) for h in range(2))

        @plsc.parallel_loop(3, _L - 3, carry=init, unroll=2)
        def c_body(c, win):
            new = (wrow(c + 3, 0), wrow(c + 3, 1))
            rows = tuple(win[2 * i:2 * i + 2] for i in range(6)) + (new,)
            for h in range(2):
                hs = pl.ds(h * 16, 16)
                pr = [rows[3 + d][h] * kv(c, d, h) for d in range(-3, 4)]
                m3 = jnp.maximum(jnp.maximum(pr[2], pr[3]), pr[4])
                m5 = jnp.maximum(m3, jnp.maximum(pr[1], pr[5]))
                m7 = jnp.maximum(m5, jnp.maximum(pr[0], pr[6]))
                o3_v[c - 1, hs] = m3
                o5_v[c - 2, hs] = m5
                o7_v[c - 3, hs] = m7
            return win[2:] + new

        for c in (1, 2, _L - 3, _L - 2):
            for h in range(2):
                hs = pl.ds(h * 16, 16)
                p0 = _p(c, 0, h)
                m3 = jnp.maximum(jnp.maximum(_p(c, -1, h), p0), _p(c, 1, h))
                o3_v[c - 1, hs] = m3
                if 2 <= c <= _L - 3:
                    m5 = jnp.maximum(
                        m3, jnp.maximum(_p(c, -2, h), _p(c, 2, h)))
                    o5_v[c - 2, hs] = m5

    issue_g(0, 0)
    wait_g(0)
    issue_g(1, 1)
    compute(0)
    issue_s(0)

    def pair_body(i, carry):
        ri = 1 + 2 * i
        for step in range(2):
            p = (1 + step) % 2
            wait_g(p)
            issue_g(ri + step + 1, 1 - p)
            wait_s()
            compute(p)
            issue_s(ri + step)
        return carry

    lax.fori_loop(0, (_ROWS_PER_W - 2) // 2, pair_body, None)

    wait_g(1)
    wait_s()
    compute(1)
    issue_s(_ROWS_PER_W - 1)
    wait_s()


@jax.jit
def _impl(seq, W, K):
    seq2 = seq.astype(jnp.int32)
    kt = jnp.transpose(K, (1, 2, 0)).reshape(_MAXW * _EMB, _VOCAB)
    wt = jnp.transpose(W, (1, 0))
    grid = (_VOCAB + _VB - 1) // _VB
    table = pl.pallas_call(
        _pack_body,
        grid=(grid,),
        in_specs=[
            pl.BlockSpec((_MAXW * _EMB, _VB), lambda i: (0, i)),
            pl.BlockSpec((_EMB, _VB), lambda i: (0, i)),
        ],
        out_specs=pl.BlockSpec((_VB, _ROW), lambda i: (i, 0)),
        out_shape=jax.ShapeDtypeStruct((_VOCAB, _ROW), jnp.float32),
    )(kt, wt)

    mesh = plsc.VectorSubcoreMesh(core_axis_name="c", subcore_axis_name="s")
    run = pl.kernel(
        _sc_body,
        mesh=mesh,
        compiler_params=pltpu.CompilerParams(use_tc_tiling_on_sc=False),
        out_type=(
            jax.ShapeDtypeStruct((_B, _L3, _EMB), jnp.float32),
            jax.ShapeDtypeStruct((_B, _L5, _EMB), jnp.float32),
            jax.ShapeDtypeStruct((_B, _L7, _EMB), jnp.float32),
        ),
        scratch_types=[
            pltpu.VMEM((_L,), jnp.int32),
            pltpu.VMEM((_L,), jnp.int32),
            pltpu.VMEM((_L, _ROW), jnp.float32),
            pltpu.VMEM((_L, _ROW), jnp.float32),
            pltpu.VMEM((_L3, _EMB), jnp.float32),
            pltpu.VMEM((_L5, _EMB), jnp.float32),
            pltpu.VMEM((_L7, _EMB), jnp.float32),
            pltpu.SemaphoreType.DMA,
            pltpu.SemaphoreType.DMA,
        ],
    )
    return run(seq2, table)


def kernel(seq, W, K):
    return _impl(seq, W, K)


# speedup vs baseline: 1.5497x; 1.1094x over previous
import jax
import jax.numpy as jnp
from jax import lax
from jax.experimental import pallas as pl
from jax.experimental.pallas import tpu as pltpu
from jax.experimental.pallas import tpu_sc as plsc

_VOCAB = 100000
_EMB = 32
_MAXW = 7
_B = 1024
_L = 200
_NCORES = 2
_NSUB = 16
_NW = _NCORES * _NSUB
_ROWS_PER_W = _B // _NW
_LP = 208
_CHUNKS = ((0, 128), (128, 128), (256, 128), (384, 16))
_L3 = _L - 2
_L5 = _L - 4
_L7 = _L - 6
_VB = 4096
_ROW = _MAXW * _EMB + _EMB


def _pack_body(kt_ref, wt_ref, out_ref):
    t = jnp.transpose(kt_ref[...], (1, 0))
    w = jnp.transpose(wt_ref[...], (1, 0))
    r1 = t[:, :128]
    r2 = jnp.concatenate([t[:, 128:], w], axis=1)
    out_ref[...] = jnp.stack([r1, r2], axis=1).reshape(2 * _VB, 128)


def _sc_body(seq_hbm, t_hbm, o3_hbm, o5_hbm, o7_hbm,
             idx0, idx1, di0, di1, dst0, dst1, o3_v, o5_v, o7_v, gsem, ssem):
    cid = lax.axis_index("c")
    sid = lax.axis_index("s")
    wid = sid * _NCORES + cid
    bufs = ((idx0, di0, dst0), (idx1, di1, dst1))

    def gather_copies(p):
        idx_v, di, dst = bufs[p]
        cps = []
        for off, n in _CHUNKS:
            sl = pl.ds(off, n)
            cps.append(pltpu.make_async_copy(
                t_hbm.at[di.at[sl]], dst.at[sl], gsem))
        return cps

    def issue_g(ri, p):
        idx_v, di, dst = bufs[p]
        b = wid * _ROWS_PER_W + ri
        pltpu.sync_copy(seq_hbm.at[b], idx_v)
        iota = lax.iota(jnp.int32, 16)
        for g in range(_LP // 16):
            v2 = idx_v[pl.ds(g * 16, 16)] * 2
            pos = iota * 2 + (g * 32)
            plsc.store_scatter(di, [pos], v2)
            plsc.store_scatter(di, [pos + 1], v2 + 1)
        for cp in gather_copies(p):
            cp.start()

    def wait_g(p):
        for cp in gather_copies(p):
            cp.wait()

    def scatter_copies(ri):
        b = wid * _ROWS_PER_W + ri
        return [pltpu.make_async_copy(o3_v, o3_hbm.at[b], ssem),
                pltpu.make_async_copy(o5_v, o5_hbm.at[b], ssem),
                pltpu.make_async_copy(o7_v, o7_hbm.at[b], ssem)]

    def issue_s(ri):
        for cp in scatter_copies(ri):
            cp.start()

    def wait_s():
        for cp in scatter_copies(0):
            cp.wait()

    def compute(p):
        idx_v, di, dst = bufs[p]

        def wrow(c, h):
            return dst[2 * c + 1, pl.ds(96 + h * 16, 16)]

        def kv(c, d, h):
            off = (3 + d) * _EMB + h * 16
            if off < 128:
                return dst[2 * c, pl.ds(off, 16)]
            return dst[2 * c + 1, pl.ds(off - 128, 16)]

        def _p(c, d, h):
            return wrow(c + d, h) * kv(c, d, h)

        init = tuple(wrow(c, h) for c in range(6) for h in range(2))

        @plsc.parallel_loop(3, _L - 3, carry=init, unroll=2)
        def c_body(c, win):
            new = (wrow(c + 3, 0), wrow(c + 3, 1))
            rows = tuple(win[2 * i:2 * i + 2] for i in range(6)) + (new,)
            for h in range(2):
                hs = pl.ds(h * 16, 16)
                pr = [rows[3 + d][h] * kv(c, d, h) for d in range(-3, 4)]
                m3 = jnp.maximum(jnp.maximum(pr[2], pr[3]), pr[4])
                m5 = jnp.maximum(m3, jnp.maximum(pr[1], pr[5]))
                m7 = jnp.maximum(m5, jnp.maximum(pr[0], pr[6]))
                o3_v[c - 1, hs] = m3
                o5_v[c - 2, hs] = m5
                o7_v[c - 3, hs] = m7
            return win[2:] + new

        for c in (1, 2, _L - 3, _L - 2):
            for h in range(2):
                hs = pl.ds(h * 16, 16)
                p0 = _p(c, 0, h)
                m3 = jnp.maximum(jnp.maximum(_p(c, -1, h), p0), _p(c, 1, h))
                o3_v[c - 1, hs] = m3
                if 2 <= c <= _L - 3:
                    m5 = jnp.maximum(
                        m3, jnp.maximum(_p(c, -2, h), _p(c, 2, h)))
                    o5_v[c - 2, hs] = m5

    issue_g(0, 0)
    wait_g(0)
    issue_g(1, 1)
    compute(0)
    issue_s(0)

    def pair_body(i, carry):
        ri = 1 + 2 * i
        for step in range(2):
            p = (1 + step) % 2
            wait_g(p)
            issue_g(ri + step + 1, 1 - p)
            wait_s()
            compute(p)
            issue_s(ri + step)
        return carry

    lax.fori_loop(0, (_ROWS_PER_W - 2) // 2, pair_body, None)

    wait_g(1)
    wait_s()
    compute(1)
    issue_s(_ROWS_PER_W - 1)
    wait_s()


@jax.jit
def _impl(seq, W, K):
    seq2 = jnp.pad(seq.astype(jnp.int32), ((0, 0), (0, _LP - _L)))
    kt = jnp.transpose(K, (1, 2, 0)).reshape(_MAXW * _EMB, _VOCAB)
    wt = jnp.transpose(W, (1, 0))
    grid = (_VOCAB + _VB - 1) // _VB
    table = pl.pallas_call(
        _pack_body,
        grid=(grid,),
        in_specs=[
            pl.BlockSpec((_MAXW * _EMB, _VB), lambda i: (0, i)),
            pl.BlockSpec((_EMB, _VB), lambda i: (0, i)),
        ],
        out_specs=pl.BlockSpec((2 * _VB, 128), lambda i: (i, 0)),
        out_shape=jax.ShapeDtypeStruct((2 * _VOCAB, 128), jnp.float32),
    )(kt, wt)

    mesh = plsc.VectorSubcoreMesh(core_axis_name="c", subcore_axis_name="s")
    run = pl.kernel(
        _sc_body,
        mesh=mesh,
        compiler_params=pltpu.CompilerParams(
            use_tc_tiling_on_sc=False, needs_layout_passes=False),
        out_type=(
            jax.ShapeDtypeStruct((_B, _L3, _EMB), jnp.float32),
            jax.ShapeDtypeStruct((_B, _L5, _EMB), jnp.float32),
            jax.ShapeDtypeStruct((_B, _L7, _EMB), jnp.float32),
        ),
        scratch_types=[
            pltpu.VMEM((_LP,), jnp.int32),
            pltpu.VMEM((_LP,), jnp.int32),
            pltpu.VMEM((2 * _LP,), jnp.int32),
            pltpu.VMEM((2 * _LP,), jnp.int32),
            pltpu.VMEM((2 * _L, 128), jnp.float32),
            pltpu.VMEM((2 * _L, 128), jnp.float32),
            pltpu.VMEM((_L3, _EMB), jnp.float32),
            pltpu.VMEM((_L5, _EMB), jnp.float32),
            pltpu.VMEM((_L7, _EMB), jnp.float32),
            pltpu.SemaphoreType.DMA,
            pltpu.SemaphoreType.DMA,
        ],
    )
    return run(seq2, table)


def kernel(seq, W, K):
    return _impl(seq, W, K)


# speedup vs baseline: 1.6088x; 1.0382x over previous
import jax
import jax.numpy as jnp
from jax import lax
from jax.experimental import pallas as pl
from jax.experimental.pallas import tpu as pltpu
from jax.experimental.pallas import tpu_sc as plsc

_VOCAB = 100000
_EMB = 32
_MAXW = 7
_B = 1024
_L = 200
_NCORES = 2
_NSUB = 16
_NW = _NCORES * _NSUB
_ROWS_PER_W = _B // _NW
_CHUNKS = ((0, 128), (128, 72))
_L3 = _L - 2
_L5 = _L - 4
_L7 = _L - 6
_VB = 6400


def _pack_body(kt_ref, wt_ref, a_ref, b_ref):
    t = jnp.transpose(kt_ref[...], (1, 0))
    w = jnp.transpose(wt_ref[...], (1, 0))
    a_ref[...] = t[:, :128]
    b_ref[...] = jnp.concatenate([t[:, 128:], w], axis=1)


def _sc_body(seq_hbm, a_hbm, b_hbm, o3_hbm, o5_hbm, o7_hbm,
             idx0, idx1, da0, da1, db0, db1, o3_v, o5_v, o7_v, gsem, ssem):
    cid = lax.axis_index("c")
    sid = lax.axis_index("s")
    wid = sid * _NCORES + cid
    bufs = ((idx0, da0, db0), (idx1, da1, db1))

    def gather_copies(p):
        idx_v, da, db = bufs[p]
        cps = []
        for off, n in _CHUNKS:
            sl = pl.ds(off, n)
            cps.append(pltpu.make_async_copy(
                a_hbm.at[idx_v.at[sl]], da.at[sl], gsem))
            cps.append(pltpu.make_async_copy(
                b_hbm.at[idx_v.at[sl]], db.at[sl], gsem))
        return cps

    def issue_g(ri, p):
        idx_v, da, db = bufs[p]
        b = wid * _ROWS_PER_W + ri
        pltpu.sync_copy(seq_hbm.at[b], idx_v)
        for cp in gather_copies(p):
            cp.start()

    def wait_g(p):
        for cp in gather_copies(p):
            cp.wait()

    def scatter_copies(ri):
        b = wid * _ROWS_PER_W + ri
        return [pltpu.make_async_copy(o3_v, o3_hbm.at[b], ssem),
                pltpu.make_async_copy(o5_v, o5_hbm.at[b], ssem),
                pltpu.make_async_copy(o7_v, o7_hbm.at[b], ssem)]

    def issue_s(ri):
        for cp in scatter_copies(ri):
            cp.start()

    def wait_s():
        for cp in scatter_copies(0):
            cp.wait()

    def compute(p):
        idx_v, da, db = bufs[p]

        def wrow(c, h):
            return db[c, pl.ds(96 + h * 16, 16)]

        def kv(c, d, h):
            off = (3 + d) * _EMB + h * 16
            if off < 128:
                return da[c, pl.ds(off, 16)]
            return db[c, pl.ds(off - 128, 16)]

        def _p(c, d, h):
            return wrow(c + d, h) * kv(c, d, h)

        init = tuple(wrow(c, h) for c in range(6) for h in range(2))

        @plsc.parallel_loop(3, _L - 3, carry=init, unroll=2)
        def c_body(c, win):
            new = (wrow(c + 3, 0), wrow(c + 3, 1))
            rows = tuple(win[2 * i:2 * i + 2] for i in range(6)) + (new,)
            for h in range(2):
                hs = pl.ds(h * 16, 16)
                pr = [rows[3 + d][h] * kv(c, d, h) for d in range(-3, 4)]
                m3 = jnp.maximum(jnp.maximum(pr[2], pr[3]), pr[4])
                m5 = jnp.maximum(m3, jnp.maximum(pr[1], pr[5]))
                m7 = jnp.maximum(m5, jnp.maximum(pr[0], pr[6]))
                o3_v[c - 1, hs] = m3
                o5_v[c - 2, hs] = m5
                o7_v[c - 3, hs] = m7
            return win[2:] + new

        for c in (1, 2, _L - 3, _L - 2):
            for h in range(2):
                hs = pl.ds(h * 16, 16)
                p0 = _p(c, 0, h)
                m3 = jnp.maximum(jnp.maximum(_p(c, -1, h), p0), _p(c, 1, h))
                o3_v[c - 1, hs] = m3
                if 2 <= c <= _L - 3:
                    m5 = jnp.maximum(
                        m3, jnp.maximum(_p(c, -2, h), _p(c, 2, h)))
                    o5_v[c - 2, hs] = m5

    issue_g(0, 0)
    wait_g(0)
    issue_g(1, 1)
    compute(0)
    issue_s(0)

    def pair_body(i, carry):
        ri = 1 + 2 * i
        for step in range(2):
            p = (1 + step) % 2
            wait_g(p)
            issue_g(ri + step + 1, 1 - p)
            wait_s()
            compute(p)
            issue_s(ri + step)
        return carry

    lax.fori_loop(0, (_ROWS_PER_W - 2) // 2, pair_body, None)

    wait_g(1)
    wait_s()
    compute(1)
    issue_s(_ROWS_PER_W - 1)
    wait_s()


@jax.jit
def _impl(seq, W, K):
    seq2 = seq.astype(jnp.int32)
    kt = jnp.transpose(K, (1, 2, 0)).reshape(_MAXW * _EMB, _VOCAB)
    wt = jnp.transpose(W, (1, 0))
    grid = (_VOCAB + _VB - 1) // _VB
    table_a, table_b = pl.pallas_call(
        _pack_body,
        grid=(grid,),
        in_specs=[
            pl.BlockSpec((_MAXW * _EMB, _VB), lambda i: (0, i)),
            pl.BlockSpec((_EMB, _VB), lambda i: (0, i)),
        ],
        out_specs=[
            pl.BlockSpec((_VB, 128), lambda i: (i, 0)),
            pl.BlockSpec((_VB, 128), lambda i: (i, 0)),
        ],
        out_shape=[
            jax.ShapeDtypeStruct((_VOCAB, 128), jnp.float32),
            jax.ShapeDtypeStruct((_VOCAB, 128), jnp.float32),
        ],
    )(kt, wt)

    mesh = plsc.VectorSubcoreMesh(core_axis_name="c", subcore_axis_name="s")
    run = pl.kernel(
        _sc_body,
        mesh=mesh,
        compiler_params=pltpu.CompilerParams(use_tc_tiling_on_sc=False),
        out_type=(
            jax.ShapeDtypeStruct((_B, _L3, _EMB), jnp.float32),
            jax.ShapeDtypeStruct((_B, _L5, _EMB), jnp.float32),
            jax.ShapeDtypeStruct((_B, _L7, _EMB), jnp.float32),
        ),
        scratch_types=[
            pltpu.VMEM((_L,), jnp.int32),
            pltpu.VMEM((_L,), jnp.int32),
            pltpu.VMEM((_L, 128), jnp.float32),
            pltpu.VMEM((_L, 128), jnp.float32),
            pltpu.VMEM((_L, 128), jnp.float32),
            pltpu.VMEM((_L, 128), jnp.float32),
            pltpu.VMEM((_L3, _EMB), jnp.float32),
            pltpu.VMEM((_L5, _EMB), jnp.float32),
            pltpu.VMEM((_L7, _EMB), jnp.float32),
            pltpu.SemaphoreType.DMA,
            pltpu.SemaphoreType.DMA,
        ],
    )
    return run(seq2, table_a, table_b)


def kernel(seq, W, K):
    return _impl(seq, W, K)
